# Initial kernel scaffold; baseline (speedup 1.0000x reference)
import math

import jax
import jax.numpy as jnp
from jax import lax
from jax.experimental import pallas as pl
from jax.experimental.pallas import tpu as pltpu
from jax.experimental.pallas import tpu_sc as plsc

_CUTOFF = 10.0
_ON_CUT = 0.25 * _CUTOFF
_OFF_CUT = 0.75 * _CUTOFF
_ALPHA = 4.0 / _CUTOFF + 0.001
_ALPHA2 = _ALPHA ** 2
_ONE_OVER_SQRTPI = 1.0 / math.sqrt(math.pi)
_NMAX = (2, 2, 2)

_N_ATOMS = 100000
_N_EDGES = 6400000
_N_MOL = 64

_NC = 2
_NS = 16
_NW = _NC * _NS
_LANES = 16

_NP = 100352
_BA = 2048
_NBLK = _NP // _BA
_CH = 2048
_CH_ROWS = 16
_NCHUNK = 98
_NE_PAD = _NW * _NCHUNK * _CH
_EROWS = _NE_PAD // 128
_ROWS_PER_W = _EROWS // _NW
_OUT_SLICE = _NP // _NS

_EA1 = 0.254829592
_EA2 = -0.284496736
_EA3 = 1.421413741
_EA4 = -1.453152027
_EA5 = 1.061405429
_EP = 0.3275911


def _kvec_mats():
    import itertools
    import numpy as np
    k = []
    for i in range(3):
        kk = [float(v) for v in range(_NMAX[i] + 1)] + [
            float(-v) for v in range(1, _NMAX[i] + 1)]
        k.append(kk)
    lk = list(itertools.product(k[0], k[1], k[2]))[1:]
    kvecs = np.array(lk, dtype=np.float32)
    kmax = max(_NMAX)
    kvecs = kvecs[(kvecs ** 2).sum(-1) <= kmax ** 2]
    return jnp.asarray(kvecs)


_KVECS = _kvec_mats()
_NK = _KVECS.shape[0]


def _pw_vector(d, qi, qj):
    one = jnp.float32(1.0)
    fac = qi * qj
    d2 = d * d
    z = jnp.float32(_ALPHA) * d
    t = one / (one + jnp.float32(_EP) * z)
    ez = jnp.exp(jnp.float32(-_ALPHA2) * d2)
    poly = t * (jnp.float32(_EA1) + t * (jnp.float32(_EA2) + t * (
        jnp.float32(_EA3) + t * (jnp.float32(_EA4) + t * jnp.float32(_EA5)))))
    erfc = poly * ez
    x = (d - jnp.float32(_ON_CUT)) * jnp.float32(1.0 / (_OFF_CUT - _ON_CUT))
    xc = jnp.minimum(jnp.maximum(x, jnp.float32(1e-6)), jnp.float32(1.0 - 1e-6))
    g = one / (one - xc) - one / xc
    g = jnp.minimum(jnp.maximum(g, jnp.float32(-60.0)), jnp.float32(60.0))
    f = one / (one + jnp.exp(g))
    f = jnp.where(x <= 0, one, jnp.where(x >= 1, jnp.float32(0.0), f))
    v = d2 + one
    i = plsc.bitcast(v, jnp.int32)
    i = jnp.int32(0x5F3759DF) - lax.shift_right_logical(i, 1)
    y = plsc.bitcast(i, jnp.float32)
    half_v = jnp.float32(0.5) * v
    for _ in range(3):
        y = y * (jnp.float32(1.5) - half_v * y * y)
    coulomb = one / d
    return jnp.float32(0.5) * fac * (f * y + (one - f) * coulomb) * erfc


def _sc_body(qa_hbm, dij_hbm, ii_hbm, jj_hbm, zeros_hbm, out_hbm,
             qa_v, acc, d_v, ii_v, jj_v, pw_v):
    cid = lax.axis_index("c")
    sid = lax.axis_index("s")
    wid = sid * _NC + cid

    pltpu.sync_copy(qa_hbm, qa_v)

    @pl.when(sid == 0)
    def _():
        pltpu.sync_copy(zeros_hbm, acc)

    plsc.subcore_barrier()

    row0 = wid * _ROWS_PER_W

    def chunk_body(ch, carry):
        base = row0 + ch * _CH_ROWS
        pltpu.sync_copy(dij_hbm.at[pl.ds(base, _CH_ROWS)], d_v)
        pltpu.sync_copy(ii_hbm.at[pl.ds(base, _CH_ROWS)], ii_v)
        pltpu.sync_copy(jj_hbm.at[pl.ds(base, _CH_ROWS)], jj_v)

        def row_body(r, carry2):
            for cc in range(128 // _LANES):
                co = cc * _LANES
                ii = ii_v[r, pl.ds(co, _LANES)]
                jj = jj_v[r, pl.ds(co, _LANES)]
                d = d_v[r, pl.ds(co, _LANES)]
                qi = plsc.load_gather(qa_v, [ii])
                qj = plsc.load_gather(qa_v, [jj])
                pw_v[r, pl.ds(co, _LANES)] = _pw_vector(d, qi, qj)
            return carry2

        lax.fori_loop(0, _CH_ROWS, row_body, 0)
        pltpu.sync_copy(pw_v, acc.at[ii_v], add=True)
        return carry

    lax.fori_loop(0, _NCHUNK, chunk_body, 0)

    plsc.subcore_barrier()

    s0 = sid * _OUT_SLICE
    pltpu.sync_copy(acc.at[pl.ds(s0, _OUT_SLICE)],
                    out_hbm.at[cid, pl.ds(s0, _OUT_SLICE)])


def _real_space_sc(qa_pad, dij2d, ii2d, jj2d):
    zeros = jnp.zeros((_NP,), jnp.float32)
    mesh = plsc.VectorSubcoreMesh(core_axis_name="c", subcore_axis_name="s")
    run = pl.kernel(
        _sc_body,
        out_type=jax.ShapeDtypeStruct((_NC, _NP), jnp.float32),
        mesh=mesh,
        scratch_types=[
            pltpu.VMEM((_NP,), jnp.float32),
            pltpu.VMEM_SHARED((_NP,), jnp.float32),
            pltpu.VMEM((_CH_ROWS, 128), jnp.float32),
            pltpu.VMEM((_CH_ROWS, 128), jnp.int32),
            pltpu.VMEM((_CH_ROWS, 128), jnp.int32),
            pltpu.VMEM((_CH_ROWS, 128), jnp.float32),
        ],
    )
    return run(qa_pad, dij2d, ii2d, jj2d, zeros)


_DN_SEG = (((0,), (0,)), ((), ()))


def _moments_body(qa_ref, rx_ref, ry_ref, rz_ref, bs_ref,
                  kx_ref, ky_ref, kz_ref,
                  qr_ref, qi_ref, misc_ref):
    pid = pl.program_id(0)
    qa = qa_ref[0, :]
    bs = bs_ref[0, :]
    oh = (bs[:, None] == lax.broadcasted_iota(jnp.int32, (_BA, _N_MOL), 1)
          ).astype(jnp.float32)
    hp = jax.lax.Precision.HIGHEST
    kdp = (jnp.dot(oh, kx_ref[...], precision=hp) * rx_ref[0, :][:, None]
           + jnp.dot(oh, ky_ref[...], precision=hp) * ry_ref[0, :][:, None]
           + jnp.dot(oh, kz_ref[...], precision=hp) * rz_ref[0, :][:, None])
    qc = qa[:, None] * jnp.cos(kdp)
    qs = qa[:, None] * jnp.sin(kdp)
    qr_c = lax.dot_general(oh, qc, _DN_SEG, precision=hp)
    qi_c = lax.dot_general(oh, qs, _DN_SEG, precision=hp)
    aidx = pid * _BA + lax.broadcasted_iota(jnp.int32, (_BA,), 0)
    valid = (aidx < _N_ATOMS).astype(jnp.float32)
    mcols = jnp.stack(
        [qa * qa, qa * rx_ref[0, :], qa * ry_ref[0, :], qa * rz_ref[0, :],
         valid, jnp.zeros_like(qa), jnp.zeros_like(qa), jnp.zeros_like(qa)],
        axis=1)
    misc_c = lax.dot_general(oh, mcols, _DN_SEG, precision=hp)

    @pl.when(pid == 0)
    def _():
        qr_ref[...] = jnp.zeros_like(qr_ref)
        qi_ref[...] = jnp.zeros_like(qi_ref)
        misc_ref[...] = jnp.zeros_like(misc_ref)

    qr_ref[...] += qr_c
    qi_ref[...] += qi_c
    misc_ref[...] += misc_c


def _recip_moments_tc(qa2d, rx2d, ry2d, rz2d, bs2d, kx, ky, kz):
    blk = pl.BlockSpec((1, _BA), lambda i: (i, 0))
    full = lambda s: pl.BlockSpec(s, lambda i: (0, 0))
    return pl.pallas_call(
        _moments_body,
        grid=(_NBLK,),
        in_specs=[blk, blk, blk, blk, blk,
                  full((_N_MOL, _NK)), full((_N_MOL, _NK)), full((_N_MOL, _NK))],
        out_specs=[full((_N_MOL, _NK)), full((_N_MOL, _NK)), full((_N_MOL, 8))],
        out_shape=[jax.ShapeDtypeStruct((_N_MOL, _NK), jnp.float32),
                   jax.ShapeDtypeStruct((_N_MOL, _NK), jnp.float32),
                   jax.ShapeDtypeStruct((_N_MOL, 8), jnp.float32)],
    )(qa2d, rx2d, ry2d, rz2d, bs2d, kx, ky, kz)


def _combine_body(p0_ref, p1_ref, bs_ref, er_ref, out_ref):
    bs = bs_ref[0, :]
    oh = (bs[:, None] == lax.broadcasted_iota(jnp.int32, (_BA, _N_MOL), 1)
          ).astype(jnp.float32)
    er_at = jnp.dot(oh, er_ref[...], precision=jax.lax.Precision.HIGHEST)
    out_ref[0, :] = p0_ref[0, :] + p1_ref[0, :] + er_at[:, 0]


def _combine_tc(p0, p1, bs2d, er_col):
    blk = pl.BlockSpec((1, _BA), lambda i: (i, 0))
    return pl.pallas_call(
        _combine_body,
        grid=(_NBLK,),
        in_specs=[blk, blk, blk, pl.BlockSpec((_N_MOL, 1), lambda i: (0, 0))],
        out_specs=blk,
        out_shape=jax.ShapeDtypeStruct((_NBLK, _BA), jnp.float32),
    )(p0, p1, bs2d, er_col)


def kernel(Z, Dij, Qa, idx_i, idx_j, R, batch_seg, Cell):
    del Z
    f32 = jnp.float32

    qa_pad = jnp.pad(Qa, (0, _NP - _N_ATOMS))
    dij2d = jnp.pad(Dij, (0, _NE_PAD - _N_EDGES),
                    constant_values=1.0).reshape(_EROWS, 128)
    ii2d = jnp.pad(idx_i, (0, _NE_PAD - _N_EDGES),
                   constant_values=_NP - 1).reshape(_EROWS, 128)
    jj2d = jnp.pad(idx_j, (0, _NE_PAD - _N_EDGES),
                   constant_values=_NP - 1).reshape(_EROWS, 128)

    qa2d = qa_pad.reshape(_NBLK, _BA)
    bs2d = jnp.pad(batch_seg, (0, _NP - _N_ATOMS)).reshape(_NBLK, _BA)
    r_pad = jnp.pad(R, ((0, _NP - _N_ATOMS), (0, 0)))
    rx2d = r_pad[:, 0].reshape(_NBLK, _BA)
    ry2d = r_pad[:, 1].reshape(_NBLK, _BA)
    rz2d = r_pad[:, 2].reshape(_NBLK, _BA)

    recip_box = 2.0 * jnp.pi * jnp.linalg.inv(Cell)
    recip_box = jnp.transpose(recip_box, (0, 2, 1))
    v_box = jnp.abs(jnp.linalg.det(Cell))
    prefactor = 2.0 * jnp.pi / v_box
    k = jnp.matmul(_KVECS, recip_box)
    k2 = jnp.sum(k * k, axis=-1)
    qg = jnp.exp(-0.25 * k2 / _ALPHA2)
    kx, ky, kz = k[..., 0], k[..., 1], k[..., 2]

    sc_out = _real_space_sc(qa_pad, dij2d, ii2d, jj2d)
    q_real, q_imag, misc = _recip_moments_tc(qa2d, rx2d, ry2d, rz2d, bs2d,
                                             kx.astype(f32), ky.astype(f32),
                                             kz.astype(f32))

    qa2_m = misc[:, 0]
    qar = misc[:, 1:4]
    na_m = misc[:, 4]
    q_dens = q_real ** 2 + q_imag ** 2
    e_rec = prefactor * jnp.sum(q_dens * qg / k2)
    e_rec = e_rec - _ALPHA * _ONE_OVER_SQRTPI * qa2_m
    e_rec = e_rec + prefactor / 3.0 * jnp.sum(qar * qar, axis=-1)
    er_col = (e_rec / na_m)[:, None].astype(f32)

    p0 = sc_out[0].reshape(_NBLK, _BA)
    p1 = sc_out[1].reshape(_NBLK, _BA)
    energy2d = _combine_tc(p0, p1, bs2d, er_col)
    energy = energy2d.reshape(_NP)[:_N_ATOMS]
    return (energy, Qa)


# speedup vs baseline: 191.1103x; 191.1103x over previous
import math

import jax
import jax.numpy as jnp
from jax import lax
from jax.experimental import pallas as pl
from jax.experimental.pallas import tpu as pltpu
from jax.experimental.pallas import tpu_sc as plsc

_CUTOFF = 10.0
_ON_CUT = 0.25 * _CUTOFF
_OFF_CUT = 0.75 * _CUTOFF
_ALPHA = 4.0 / _CUTOFF + 0.001
_ALPHA2 = _ALPHA ** 2
_ONE_OVER_SQRTPI = 1.0 / math.sqrt(math.pi)
_NMAX = (2, 2, 2)

_N_ATOMS = 100000
_N_EDGES = 6400000
_N_MOL = 64

_NC = 2
_NS = 16
_NW = _NC * _NS
_LANES = 16

_NP = 100352
_BA = 2048
_NBLK = _NP // _BA
_CH = 2048
_CH_ROWS = 16
_NCHUNK = 98
_NE_PAD = _NW * _NCHUNK * _CH
_EROWS = _NE_PAD // 128
_ROWS_PER_W = _EROWS // _NW
_OUT_SLICE = _NP // _NS

_EA1 = 0.254829592
_EA2 = -0.284496736
_EA3 = 1.421413741
_EA4 = -1.453152027
_EA5 = 1.061405429
_EP = 0.3275911


def _kvec_mats():
    import itertools
    import numpy as np
    k = []
    for i in range(3):
        kk = [float(v) for v in range(_NMAX[i] + 1)] + [
            float(-v) for v in range(1, _NMAX[i] + 1)]
        k.append(kk)
    lk = list(itertools.product(k[0], k[1], k[2]))[1:]
    kvecs = np.array(lk, dtype=np.float32)
    kmax = max(_NMAX)
    kvecs = kvecs[(kvecs ** 2).sum(-1) <= kmax ** 2]
    return kvecs


_KVECS = _kvec_mats()
_NK = _KVECS.shape[0]


def _pw_vector(d, qi, qj):
    one = jnp.float32(1.0)
    fac = qi * qj
    d2 = d * d
    z = jnp.float32(_ALPHA) * d
    t = one / (one + jnp.float32(_EP) * z)
    ez = jnp.exp(jnp.float32(-_ALPHA2) * d2)
    poly = t * (jnp.float32(_EA1) + t * (jnp.float32(_EA2) + t * (
        jnp.float32(_EA3) + t * (jnp.float32(_EA4) + t * jnp.float32(_EA5)))))
    erfc = poly * ez
    x = (d - jnp.float32(_ON_CUT)) * jnp.float32(1.0 / (_OFF_CUT - _ON_CUT))
    xc = jnp.minimum(jnp.maximum(x, jnp.float32(1e-6)), jnp.float32(1.0 - 1e-6))
    g = one / (one - xc) - one / xc
    g = jnp.minimum(jnp.maximum(g, jnp.float32(-60.0)), jnp.float32(60.0))
    f = one / (one + jnp.exp(g))
    f = jnp.where(x <= 0, one, jnp.where(x >= 1, jnp.float32(0.0), f))
    v = d2 + one
    i = plsc.bitcast(v, jnp.int32)
    i = jnp.int32(0x5F3759DF) - lax.shift_right_logical(i, 1)
    y = plsc.bitcast(i, jnp.float32)
    half_v = jnp.float32(0.5) * v
    for _ in range(3):
        y = y * (jnp.float32(1.5) - half_v * y * y)
    coulomb = one / d
    return jnp.float32(0.5) * fac * (f * y + (one - f) * coulomb) * erfc


def _sc_body(qa_hbm, dij_hbm, ii_hbm, jj_hbm, zeros_hbm, out_hbm,
             qa_v, acc, d_v, ii_v, jj_v, pw_v):
    cid = lax.axis_index("c")
    sid = lax.axis_index("s")
    wid = sid * _NC + cid

    pltpu.sync_copy(qa_hbm, qa_v)

    @pl.when(sid == 0)
    def _():
        pltpu.sync_copy(zeros_hbm, acc)

    plsc.subcore_barrier()

    e0 = wid * _NCHUNK * _CH

    def chunk_body(ch, carry):
        base = e0 + ch * _CH
        pltpu.sync_copy(dij_hbm.at[pl.ds(base, _CH)], d_v)
        pltpu.sync_copy(ii_hbm.at[pl.ds(base, _CH)], ii_v)
        pltpu.sync_copy(jj_hbm.at[pl.ds(base, _CH)], jj_v)

        def vec_body(t, carry2):
            co = t * _LANES
            ii = ii_v[pl.ds(co, _LANES)]
            jj = jj_v[pl.ds(co, _LANES)]
            d = d_v[pl.ds(co, _LANES)]
            qi = plsc.load_gather(qa_v, [ii])
            qj = plsc.load_gather(qa_v, [jj])
            pw_v[pl.ds(co, _LANES)] = _pw_vector(d, qi, qj)
            return carry2

        lax.fori_loop(0, _CH // _LANES, vec_body, 0)
        pltpu.sync_copy(pw_v, acc.at[ii_v], add=True)
        return carry

    lax.fori_loop(0, _NCHUNK, chunk_body, 0)

    plsc.subcore_barrier()

    s0 = sid * _OUT_SLICE
    pltpu.sync_copy(acc.at[pl.ds(s0, _OUT_SLICE)],
                    out_hbm.at[cid, pl.ds(s0, _OUT_SLICE)])


def _real_space_sc(qa_pad, dij2d, ii2d, jj2d):
    zeros = jnp.zeros((_NP,), jnp.float32)
    mesh = plsc.VectorSubcoreMesh(core_axis_name="c", subcore_axis_name="s")
    run = pl.kernel(
        _sc_body,
        out_type=jax.ShapeDtypeStruct((_NC, _NP), jnp.float32),
        mesh=mesh,
        compiler_params=pltpu.CompilerParams(
            needs_layout_passes=False, use_tc_tiling_on_sc=False),
        scratch_types=[
            pltpu.VMEM((_NP,), jnp.float32),
            pltpu.VMEM_SHARED((_NP,), jnp.float32),
            pltpu.VMEM((_CH,), jnp.float32),
            pltpu.VMEM((_CH,), jnp.int32),
            pltpu.VMEM((_CH,), jnp.int32),
            pltpu.VMEM((_CH,), jnp.float32),
        ],
    )
    return run(qa_pad, dij2d, ii2d, jj2d, zeros)


_DN_SEG = (((0,), (0,)), ((), ()))


def _moments_body(qa_ref, rx_ref, ry_ref, rz_ref, bs_ref,
                  kx_ref, ky_ref, kz_ref,
                  qr_ref, qi_ref, misc_ref):
    pid = pl.program_id(0)
    qa = qa_ref[0, 0, :]
    bs = bs_ref[0, 0, :]
    oh = (bs[:, None] == lax.broadcasted_iota(jnp.int32, (_BA, _N_MOL), 1)
          ).astype(jnp.float32)
    hp = jax.lax.Precision.HIGHEST
    kdp = (jnp.dot(oh, kx_ref[...], precision=hp) * rx_ref[0, 0, :][:, None]
           + jnp.dot(oh, ky_ref[...], precision=hp) * ry_ref[0, 0, :][:, None]
           + jnp.dot(oh, kz_ref[...], precision=hp) * rz_ref[0, 0, :][:, None])
    qc = qa[:, None] * jnp.cos(kdp)
    qs = qa[:, None] * jnp.sin(kdp)
    qr_c = lax.dot_general(oh, qc, _DN_SEG, precision=hp)
    qi_c = lax.dot_general(oh, qs, _DN_SEG, precision=hp)
    aidx = pid * _BA + lax.broadcasted_iota(jnp.int32, (_BA,), 0)
    valid = (aidx < _N_ATOMS).astype(jnp.float32)
    mcols = jnp.stack(
        [qa * qa, qa * rx_ref[0, 0, :], qa * ry_ref[0, 0, :], qa * rz_ref[0, 0, :],
         valid, jnp.zeros_like(qa), jnp.zeros_like(qa), jnp.zeros_like(qa)],
        axis=1)
    misc_c = lax.dot_general(oh, mcols, _DN_SEG, precision=hp)

    @pl.when(pid == 0)
    def _():
        qr_ref[...] = jnp.zeros_like(qr_ref)
        qi_ref[...] = jnp.zeros_like(qi_ref)
        misc_ref[...] = jnp.zeros_like(misc_ref)

    qr_ref[...] += qr_c
    qi_ref[...] += qi_c
    misc_ref[...] += misc_c


def _recip_moments_tc(qa2d, rx2d, ry2d, rz2d, bs2d, kx, ky, kz):
    blk = pl.BlockSpec((1, 1, _BA), lambda i: (i, 0, 0))
    full = lambda s: pl.BlockSpec(s, lambda i: (0, 0))
    return pl.pallas_call(
        _moments_body,
        grid=(_NBLK,),
        in_specs=[blk, blk, blk, blk, blk,
                  full((_N_MOL, _NK)), full((_N_MOL, _NK)), full((_N_MOL, _NK))],
        out_specs=[full((_N_MOL, _NK)), full((_N_MOL, _NK)), full((_N_MOL, 8))],
        out_shape=[jax.ShapeDtypeStruct((_N_MOL, _NK), jnp.float32),
                   jax.ShapeDtypeStruct((_N_MOL, _NK), jnp.float32),
                   jax.ShapeDtypeStruct((_N_MOL, 8), jnp.float32)],
    )(qa2d, rx2d, ry2d, rz2d, bs2d, kx, ky, kz)


def _combine_body(p0_ref, p1_ref, bs_ref, er_ref, out_ref):
    bs = bs_ref[0, 0, :]
    oh = (bs[:, None] == lax.broadcasted_iota(jnp.int32, (_BA, _N_MOL), 1)
          ).astype(jnp.float32)
    er_at = jnp.dot(oh, er_ref[...], precision=jax.lax.Precision.HIGHEST)
    out_ref[0, 0, :] = p0_ref[0, 0, :] + p1_ref[0, 0, :] + er_at[:, 0]


def _combine_tc(p0, p1, bs2d, er_col):
    blk = pl.BlockSpec((1, 1, _BA), lambda i: (i, 0, 0))
    return pl.pallas_call(
        _combine_body,
        grid=(_NBLK,),
        in_specs=[blk, blk, blk, pl.BlockSpec((_N_MOL, 1), lambda i: (0, 0))],
        out_specs=blk,
        out_shape=jax.ShapeDtypeStruct((_NBLK, 1, _BA), jnp.float32),
    )(p0, p1, bs2d, er_col)


def kernel(Z, Dij, Qa, idx_i, idx_j, R, batch_seg, Cell):
    del Z
    f32 = jnp.float32

    qa_pad = jnp.pad(Qa, (0, _NP - _N_ATOMS))
    dij2d = jnp.pad(Dij, (0, _NE_PAD - _N_EDGES), constant_values=1.0)
    ii2d = jnp.pad(idx_i, (0, _NE_PAD - _N_EDGES), constant_values=_NP - 1)
    jj2d = jnp.pad(idx_j, (0, _NE_PAD - _N_EDGES), constant_values=_NP - 1)

    qa2d = qa_pad.reshape(_NBLK, 1, _BA)
    bs2d = jnp.pad(batch_seg, (0, _NP - _N_ATOMS)).reshape(_NBLK, 1, _BA)
    r_pad = jnp.pad(R, ((0, _NP - _N_ATOMS), (0, 0)))
    rx2d = r_pad[:, 0].reshape(_NBLK, 1, _BA)
    ry2d = r_pad[:, 1].reshape(_NBLK, 1, _BA)
    rz2d = r_pad[:, 2].reshape(_NBLK, 1, _BA)

    recip_box = 2.0 * jnp.pi * jnp.linalg.inv(Cell)
    recip_box = jnp.transpose(recip_box, (0, 2, 1))
    v_box = jnp.abs(jnp.linalg.det(Cell))
    prefactor = 2.0 * jnp.pi / v_box
    k = jnp.matmul(jnp.asarray(_KVECS), recip_box)
    k2 = jnp.sum(k * k, axis=-1)
    qg = jnp.exp(-0.25 * k2 / _ALPHA2)
    kx, ky, kz = k[..., 0], k[..., 1], k[..., 2]

    sc_out = _real_space_sc(qa_pad, dij2d, ii2d, jj2d)
    q_real, q_imag, misc = _recip_moments_tc(qa2d, rx2d, ry2d, rz2d, bs2d,
                                             kx.astype(f32), ky.astype(f32),
                                             kz.astype(f32))

    qa2_m = misc[:, 0]
    qar = misc[:, 1:4]
    na_m = misc[:, 4]
    q_dens = q_real ** 2 + q_imag ** 2
    e_rec = prefactor * jnp.sum(q_dens * qg / k2)
    e_rec = e_rec - _ALPHA * _ONE_OVER_SQRTPI * qa2_m
    e_rec = e_rec + prefactor / 3.0 * jnp.sum(qar * qar, axis=-1)
    er_col = (e_rec / na_m)[:, None].astype(f32)

    p0 = sc_out[0].reshape(_NBLK, 1, _BA)
    p1 = sc_out[1].reshape(_NBLK, 1, _BA)
    energy2d = _combine_tc(p0, p1, bs2d, er_col)
    energy = energy2d.reshape(_NP)[:_N_ATOMS]
    return (energy, Qa)


# speedup vs baseline: 210.8926x; 1.1035x over previous
import math

import jax
import jax.numpy as jnp
from jax import lax
from jax.experimental import pallas as pl
from jax.experimental.pallas import tpu as pltpu
from jax.experimental.pallas import tpu_sc as plsc

_CUTOFF = 10.0
_ON_CUT = 0.25 * _CUTOFF
_OFF_CUT = 0.75 * _CUTOFF
_ALPHA = 4.0 / _CUTOFF + 0.001
_ALPHA2 = _ALPHA ** 2
_ONE_OVER_SQRTPI = 1.0 / math.sqrt(math.pi)
_NMAX = (2, 2, 2)

_N_ATOMS = 100000
_N_EDGES = 6400000
_N_MOL = 64

_NC = 2
_NS = 16
_NW = _NC * _NS
_LANES = 16

_NP = 100352
_BA = 2048
_NBLK = _NP // _BA
_CH = 2048
_EPW = _N_EDGES // _NW
_NFULL = _EPW // _CH
_CH_T = _EPW - _NFULL * _CH
_OUT_SLICE = _NP // _NS

_EA1 = 0.254829592
_EA2 = -0.284496736
_EA3 = 1.421413741
_EA4 = -1.453152027
_EA5 = 1.061405429
_EP = 0.3275911


def _kvec_mats():
    import itertools
    import numpy as np
    k = []
    for i in range(3):
        kk = [float(v) for v in range(_NMAX[i] + 1)] + [
            float(-v) for v in range(1, _NMAX[i] + 1)]
        k.append(kk)
    lk = list(itertools.product(k[0], k[1], k[2]))[1:]
    kvecs = np.array(lk, dtype=np.float32)
    kmax = max(_NMAX)
    kvecs = kvecs[(kvecs ** 2).sum(-1) <= kmax ** 2]
    return kvecs


_KVECS = _kvec_mats()
_NK = _KVECS.shape[0]


def _pw_vector(d, qi, qj):
    one = jnp.float32(1.0)
    fac = qi * qj
    d2 = d * d
    z = jnp.float32(_ALPHA) * d
    t = one / (one + jnp.float32(_EP) * z)
    ez = jnp.exp(jnp.float32(-_ALPHA2) * d2)
    poly = t * (jnp.float32(_EA1) + t * (jnp.float32(_EA2) + t * (
        jnp.float32(_EA3) + t * (jnp.float32(_EA4) + t * jnp.float32(_EA5)))))
    erfc = poly * ez
    x = (d - jnp.float32(_ON_CUT)) * jnp.float32(1.0 / (_OFF_CUT - _ON_CUT))
    xc = jnp.minimum(jnp.maximum(x, jnp.float32(1e-6)), jnp.float32(1.0 - 1e-6))
    g = one / (one - xc) - one / xc
    g = jnp.minimum(jnp.maximum(g, jnp.float32(-60.0)), jnp.float32(60.0))
    f = one / (one + jnp.exp(g))
    v = d2 + one
    i = plsc.bitcast(v, jnp.int32)
    i = jnp.int32(0x5F3759DF) - lax.shift_right_logical(i, 1)
    y = plsc.bitcast(i, jnp.float32)
    half_v = jnp.float32(0.5) * v
    for _ in range(3):
        y = y * (jnp.float32(1.5) - half_v * y * y)
    coulomb = one / d
    return jnp.float32(0.5) * fac * (f * y + (one - f) * coulomb) * erfc


def _compute_chunk(qa_v, d_v, ii_v, jj_v, pw_v, nvec):
    def vec_body(t, carry):
        co = t * _LANES
        ii = ii_v[pl.ds(co, _LANES)]
        jj = jj_v[pl.ds(co, _LANES)]
        d = d_v[pl.ds(co, _LANES)]
        qi = plsc.load_gather(qa_v, [ii])
        qj = plsc.load_gather(qa_v, [jj])
        pw_v[pl.ds(co, _LANES)] = _pw_vector(d, qi, qj)
        return carry

    lax.fori_loop(0, nvec, vec_body, 0)


def _sc_body(qa_hbm, dij_hbm, ii_hbm, jj_hbm, zeros_hbm, out_hbm,
             qa_v, acc,
             d0_v, i0_v, j0_v, d1_v, i1_v, j1_v, pw_v,
             dt_v, it_v, jt_v, pwt_v,
             sem0, sem1):
    cid = lax.axis_index("c")
    sid = lax.axis_index("s")
    wid = sid * _NC + cid

    pltpu.sync_copy(qa_hbm, qa_v)

    @pl.when(sid == 0)
    def _():
        pltpu.sync_copy(zeros_hbm, acc)

    plsc.subcore_barrier()

    e0 = wid * _EPW
    bufs = ((d0_v, i0_v, j0_v, sem0), (d1_v, i1_v, j1_v, sem1))

    def start_loads(ch, buf):
        dv, iv, jv, sem = buf
        base = e0 + ch * _CH
        pltpu.async_copy(dij_hbm.at[pl.ds(base, _CH)], dv, sem)
        pltpu.async_copy(ii_hbm.at[pl.ds(base, _CH)], iv, sem)
        pltpu.async_copy(jj_hbm.at[pl.ds(base, _CH)], jv, sem)

    def wait_loads(buf):
        dv, iv, jv, sem = buf
        pltpu.make_async_copy(dij_hbm.at[pl.ds(0, _CH)], dv, sem).wait()
        pltpu.make_async_copy(ii_hbm.at[pl.ds(0, _CH)], iv, sem).wait()
        pltpu.make_async_copy(jj_hbm.at[pl.ds(0, _CH)], jv, sem).wait()

    def do_chunk(buf):
        dv, iv, jv, _ = buf
        wait_loads(buf)
        _compute_chunk(qa_v, dv, iv, jv, pw_v, _CH // _LANES)
        pltpu.sync_copy(pw_v, acc.at[iv], add=True)

    start_loads(0, bufs[0])

    def super_body(i, carry):
        ch = i * 2
        start_loads(ch + 1, bufs[1])
        do_chunk(bufs[0])
        start_loads(ch + 2, bufs[0])
        do_chunk(bufs[1])
        return carry

    lax.fori_loop(0, (_NFULL - 1) // 2, super_body, 0)
    do_chunk(bufs[0])

    tbase = e0 + _NFULL * _CH
    pltpu.sync_copy(dij_hbm.at[pl.ds(tbase, _CH_T)], dt_v)
    pltpu.sync_copy(ii_hbm.at[pl.ds(tbase, _CH_T)], it_v)
    pltpu.sync_copy(jj_hbm.at[pl.ds(tbase, _CH_T)], jt_v)
    _compute_chunk(qa_v, dt_v, it_v, jt_v, pwt_v, _CH_T // _LANES)
    pltpu.sync_copy(pwt_v, acc.at[it_v], add=True)

    plsc.subcore_barrier()

    s0 = sid * _OUT_SLICE
    pltpu.sync_copy(acc.at[pl.ds(s0, _OUT_SLICE)],
                    out_hbm.at[cid, pl.ds(s0, _OUT_SLICE)])


def _real_space_sc(qa_pad, dij2d, ii2d, jj2d):
    zeros = jnp.zeros((_NP,), jnp.float32)
    mesh = plsc.VectorSubcoreMesh(core_axis_name="c", subcore_axis_name="s")
    run = pl.kernel(
        _sc_body,
        out_type=jax.ShapeDtypeStruct((_NC, _NP), jnp.float32),
        mesh=mesh,
        compiler_params=pltpu.CompilerParams(
            needs_layout_passes=False, use_tc_tiling_on_sc=False),
        scratch_types=[
            pltpu.VMEM((_NP,), jnp.float32),
            pltpu.VMEM_SHARED((_NP,), jnp.float32),
            pltpu.VMEM((_CH,), jnp.float32),
            pltpu.VMEM((_CH,), jnp.int32),
            pltpu.VMEM((_CH,), jnp.int32),
            pltpu.VMEM((_CH,), jnp.float32),
            pltpu.VMEM((_CH,), jnp.int32),
            pltpu.VMEM((_CH,), jnp.int32),
            pltpu.VMEM((_CH,), jnp.float32),
            pltpu.VMEM((_CH_T,), jnp.float32),
            pltpu.VMEM((_CH_T,), jnp.int32),
            pltpu.VMEM((_CH_T,), jnp.int32),
            pltpu.VMEM((_CH_T,), jnp.float32),
            pltpu.SemaphoreType.DMA,
            pltpu.SemaphoreType.DMA,
        ],
    )
    return run(qa_pad, dij2d, ii2d, jj2d, zeros)


_DN_SEG = (((0,), (0,)), ((), ()))


def _moments_body(qa_ref, rx_ref, ry_ref, rz_ref, bs_ref,
                  kx_ref, ky_ref, kz_ref,
                  qr_ref, qi_ref, misc_ref):
    pid = pl.program_id(0)
    qa = qa_ref[0, 0, :]
    bs = bs_ref[0, 0, :]
    oh = (bs[:, None] == lax.broadcasted_iota(jnp.int32, (_BA, _N_MOL), 1)
          ).astype(jnp.float32)
    hp = jax.lax.Precision.HIGHEST
    kdp = (jnp.dot(oh, kx_ref[...], precision=hp) * rx_ref[0, 0, :][:, None]
           + jnp.dot(oh, ky_ref[...], precision=hp) * ry_ref[0, 0, :][:, None]
           + jnp.dot(oh, kz_ref[...], precision=hp) * rz_ref[0, 0, :][:, None])
    qc = qa[:, None] * jnp.cos(kdp)
    qs = qa[:, None] * jnp.sin(kdp)
    qr_c = lax.dot_general(oh, qc, _DN_SEG, precision=hp)
    qi_c = lax.dot_general(oh, qs, _DN_SEG, precision=hp)
    aidx = pid * _BA + lax.broadcasted_iota(jnp.int32, (_BA,), 0)
    valid = (aidx < _N_ATOMS).astype(jnp.float32)
    mcols = jnp.stack(
        [qa * qa, qa * rx_ref[0, 0, :], qa * ry_ref[0, 0, :], qa * rz_ref[0, 0, :],
         valid, jnp.zeros_like(qa), jnp.zeros_like(qa), jnp.zeros_like(qa)],
        axis=1)
    misc_c = lax.dot_general(oh, mcols, _DN_SEG, precision=hp)

    @pl.when(pid == 0)
    def _():
        qr_ref[...] = jnp.zeros_like(qr_ref)
        qi_ref[...] = jnp.zeros_like(qi_ref)
        misc_ref[...] = jnp.zeros_like(misc_ref)

    qr_ref[...] += qr_c
    qi_ref[...] += qi_c
    misc_ref[...] += misc_c


def _recip_moments_tc(qa2d, rx2d, ry2d, rz2d, bs2d, kx, ky, kz):
    blk = pl.BlockSpec((1, 1, _BA), lambda i: (i, 0, 0))
    full = lambda s: pl.BlockSpec(s, lambda i: (0, 0))
    return pl.pallas_call(
        _moments_body,
        grid=(_NBLK,),
        in_specs=[blk, blk, blk, blk, blk,
                  full((_N_MOL, _NK)), full((_N_MOL, _NK)), full((_N_MOL, _NK))],
        out_specs=[full((_N_MOL, _NK)), full((_N_MOL, _NK)), full((_N_MOL, 8))],
        out_shape=[jax.ShapeDtypeStruct((_N_MOL, _NK), jnp.float32),
                   jax.ShapeDtypeStruct((_N_MOL, _NK), jnp.float32),
                   jax.ShapeDtypeStruct((_N_MOL, 8), jnp.float32)],
    )(qa2d, rx2d, ry2d, rz2d, bs2d, kx, ky, kz)


def _combine_body(p0_ref, p1_ref, bs_ref, er_ref, out_ref):
    bs = bs_ref[0, 0, :]
    oh = (bs[:, None] == lax.broadcasted_iota(jnp.int32, (_BA, _N_MOL), 1)
          ).astype(jnp.float32)
    er_at = jnp.dot(oh, er_ref[...], precision=jax.lax.Precision.HIGHEST)
    out_ref[0, 0, :] = p0_ref[0, 0, :] + p1_ref[0, 0, :] + er_at[:, 0]


def _combine_tc(p0, p1, bs2d, er_col):
    blk = pl.BlockSpec((1, 1, _BA), lambda i: (i, 0, 0))
    return pl.pallas_call(
        _combine_body,
        grid=(_NBLK,),
        in_specs=[blk, blk, blk, pl.BlockSpec((_N_MOL, 1), lambda i: (0, 0))],
        out_specs=blk,
        out_shape=jax.ShapeDtypeStruct((_NBLK, 1, _BA), jnp.float32),
    )(p0, p1, bs2d, er_col)


def kernel(Z, Dij, Qa, idx_i, idx_j, R, batch_seg, Cell):
    del Z
    f32 = jnp.float32

    qa_pad = jnp.pad(Qa, (0, _NP - _N_ATOMS))

    qa2d = qa_pad.reshape(_NBLK, 1, _BA)
    bs2d = jnp.pad(batch_seg, (0, _NP - _N_ATOMS)).reshape(_NBLK, 1, _BA)
    r_pad = jnp.pad(R, ((0, _NP - _N_ATOMS), (0, 0)))
    rx2d = r_pad[:, 0].reshape(_NBLK, 1, _BA)
    ry2d = r_pad[:, 1].reshape(_NBLK, 1, _BA)
    rz2d = r_pad[:, 2].reshape(_NBLK, 1, _BA)

    recip_box = 2.0 * jnp.pi * jnp.linalg.inv(Cell)
    recip_box = jnp.transpose(recip_box, (0, 2, 1))
    v_box = jnp.abs(jnp.linalg.det(Cell))
    prefactor = 2.0 * jnp.pi / v_box
    k = jnp.matmul(jnp.asarray(_KVECS), recip_box)
    k2 = jnp.sum(k * k, axis=-1)
    qg = jnp.exp(-0.25 * k2 / _ALPHA2)
    kx, ky, kz = k[..., 0], k[..., 1], k[..., 2]

    sc_out = _real_space_sc(qa_pad, Dij, idx_i, idx_j)
    q_real, q_imag, misc = _recip_moments_tc(qa2d, rx2d, ry2d, rz2d, bs2d,
                                             kx.astype(f32), ky.astype(f32),
                                             kz.astype(f32))

    qa2_m = misc[:, 0]
    qar = misc[:, 1:4]
    na_m = misc[:, 4]
    q_dens = q_real ** 2 + q_imag ** 2
    e_rec = prefactor * jnp.sum(q_dens * qg / k2)
    e_rec = e_rec - _ALPHA * _ONE_OVER_SQRTPI * qa2_m
    e_rec = e_rec + prefactor / 3.0 * jnp.sum(qar * qar, axis=-1)
    er_col = (e_rec / na_m)[:, None].astype(f32)

    p0 = sc_out[0].reshape(_NBLK, 1, _BA)
    p1 = sc_out[1].reshape(_NBLK, 1, _BA)
    energy2d = _combine_tc(p0, p1, bs2d, er_col)
    energy = energy2d.reshape(_NP)[:_N_ATOMS]
    return (energy, Qa)


# speedup vs baseline: 267.4563x; 1.2682x over previous
import math

import jax
import jax.numpy as jnp
from jax import lax
from jax.experimental import pallas as pl
from jax.experimental.pallas import tpu as pltpu
from jax.experimental.pallas import tpu_sc as plsc

_CUTOFF = 10.0
_ON_CUT = 0.25 * _CUTOFF
_OFF_CUT = 0.75 * _CUTOFF
_ALPHA = 4.0 / _CUTOFF + 0.001
_ALPHA2 = _ALPHA ** 2
_ONE_OVER_SQRTPI = 1.0 / math.sqrt(math.pi)
_NMAX = (2, 2, 2)

_N_ATOMS = 100000
_N_EDGES = 6400000
_N_MOL = 64

_NC = 2
_NS = 16
_NW = _NC * _NS
_LANES = 16

_NP = 100352
_BA = 2048
_NBLK = _NP // _BA
_CH = 2048
_EPW = _N_EDGES // _NW
_NFULL = _EPW // _CH
_CH_T = _EPW - _NFULL * _CH
_OUT_SLICE = _NP // _NS

_MTAB = 4096
_HTAB = 9.8 / (_MTAB - 1)


def _kvec_mats():
    import itertools
    import numpy as np
    k = []
    for i in range(3):
        kk = [float(v) for v in range(_NMAX[i] + 1)] + [
            float(-v) for v in range(1, _NMAX[i] + 1)]
        k.append(kk)
    lk = list(itertools.product(k[0], k[1], k[2]))[1:]
    kvecs = np.array(lk, dtype=np.float32)
    kmax = max(_NMAX)
    kvecs = kvecs[(kvecs ** 2).sum(-1) <= kmax ** 2]
    return kvecs


_KVECS = _kvec_mats()
_NK = _KVECS.shape[0]


def _radial_weight_table():
    d = jnp.float32(0.2) + _HTAB * jnp.arange(_MTAB, dtype=jnp.float32)
    x = (d - _ON_CUT) / (_OFF_CUT - _ON_CUT)
    ones = jnp.ones_like(x)
    zeros = jnp.zeros_like(x)
    x_p = jnp.where(x <= 0, ones, x)
    fp = jnp.where(x <= 0, zeros, jnp.exp(-ones / x_p))
    x_m = jnp.where(1.0 - x <= 0, ones, 1.0 - x)
    fm = jnp.where(1.0 - x <= 0, zeros, jnp.exp(-ones / x_m))
    f = jnp.where(x <= 0, ones, jnp.where(x >= 1, zeros, fm / (fp + fm)))
    coulomb = 1.0 / d
    damped = 1.0 / jnp.sqrt(d * d + 1.0)
    w = 0.5 * (f * damped + (1.0 - f) * coulomb) * jax.scipy.special.erfc(
        jnp.float32(_ALPHA) * d)
    return w.astype(jnp.float32)


def _compute_chunk(qa_v, wt_v, d_v, ii_v, jj_v, pw_v, nvec):
    inv_h = jnp.float32(1.0 / _HTAB)
    d0 = jnp.float32(0.2)
    imax = jnp.int32(_MTAB - 2)

    def vec_body(t, carry):
        co = t * _LANES
        ii = ii_v[pl.ds(co, _LANES)]
        jj = jj_v[pl.ds(co, _LANES)]
        d = d_v[pl.ds(co, _LANES)]
        qi = plsc.load_gather(qa_v, [ii])
        qj = plsc.load_gather(qa_v, [jj])
        x = (d - d0) * inv_h
        i0 = jnp.minimum(x.astype(jnp.int32), imax)
        frac = x - i0.astype(jnp.float32)
        w0 = plsc.load_gather(wt_v, [i0])
        w1 = plsc.load_gather(wt_v, [i0 + 1])
        w = w0 + frac * (w1 - w0)
        pw_v[pl.ds(co, _LANES)] = (qi * qj) * w
        return carry

    lax.fori_loop(0, nvec, vec_body, 0)


def _sc_body(qa_hbm, dij_hbm, ii_hbm, jj_hbm, zeros_hbm, wtab_hbm, out_hbm,
             qa_v, wt_v, acc,
             d0_v, i0_v, j0_v, d1_v, i1_v, j1_v, pw_v,
             dt_v, it_v, jt_v, pwt_v,
             sem0, sem1):
    cid = lax.axis_index("c")
    sid = lax.axis_index("s")
    wid = sid * _NC + cid

    pltpu.sync_copy(qa_hbm, qa_v)
    pltpu.sync_copy(wtab_hbm, wt_v)

    @pl.when(sid == 0)
    def _():
        pltpu.sync_copy(zeros_hbm, acc)

    plsc.subcore_barrier()

    e0 = wid * _EPW
    bufs = ((d0_v, i0_v, j0_v, sem0), (d1_v, i1_v, j1_v, sem1))

    def start_loads(ch, buf):
        dv, iv, jv, sem = buf
        base = e0 + ch * _CH
        pltpu.async_copy(dij_hbm.at[pl.ds(base, _CH)], dv, sem)
        pltpu.async_copy(ii_hbm.at[pl.ds(base, _CH)], iv, sem)
        pltpu.async_copy(jj_hbm.at[pl.ds(base, _CH)], jv, sem)

    def wait_loads(buf):
        dv, iv, jv, sem = buf
        pltpu.make_async_copy(dij_hbm.at[pl.ds(0, _CH)], dv, sem).wait()
        pltpu.make_async_copy(ii_hbm.at[pl.ds(0, _CH)], iv, sem).wait()
        pltpu.make_async_copy(jj_hbm.at[pl.ds(0, _CH)], jv, sem).wait()

    def do_chunk(buf):
        dv, iv, jv, _ = buf
        wait_loads(buf)
        _compute_chunk(qa_v, wt_v, dv, iv, jv, pw_v, _CH // _LANES)
        pltpu.sync_copy(pw_v, acc.at[iv], add=True)

    start_loads(0, bufs[0])

    def super_body(i, carry):
        ch = i * 2
        start_loads(ch + 1, bufs[1])
        do_chunk(bufs[0])
        start_loads(ch + 2, bufs[0])
        do_chunk(bufs[1])
        return carry

    lax.fori_loop(0, (_NFULL - 1) // 2, super_body, 0)
    do_chunk(bufs[0])

    tbase = e0 + _NFULL * _CH
    pltpu.sync_copy(dij_hbm.at[pl.ds(tbase, _CH_T)], dt_v)
    pltpu.sync_copy(ii_hbm.at[pl.ds(tbase, _CH_T)], it_v)
    pltpu.sync_copy(jj_hbm.at[pl.ds(tbase, _CH_T)], jt_v)
    _compute_chunk(qa_v, wt_v, dt_v, it_v, jt_v, pwt_v, _CH_T // _LANES)
    pltpu.sync_copy(pwt_v, acc.at[it_v], add=True)

    plsc.subcore_barrier()

    s0 = sid * _OUT_SLICE
    pltpu.sync_copy(acc.at[pl.ds(s0, _OUT_SLICE)],
                    out_hbm.at[cid, pl.ds(s0, _OUT_SLICE)])


def _real_space_sc(qa_pad, dij2d, ii2d, jj2d):
    zeros = jnp.zeros((_NP,), jnp.float32)
    wtab = _radial_weight_table()
    mesh = plsc.VectorSubcoreMesh(core_axis_name="c", subcore_axis_name="s")
    run = pl.kernel(
        _sc_body,
        out_type=jax.ShapeDtypeStruct((_NC, _NP), jnp.float32),
        mesh=mesh,
        compiler_params=pltpu.CompilerParams(
            needs_layout_passes=False, use_tc_tiling_on_sc=False),
        scratch_types=[
            pltpu.VMEM((_NP,), jnp.float32),
            pltpu.VMEM((_MTAB,), jnp.float32),
            pltpu.VMEM_SHARED((_NP,), jnp.float32),
            pltpu.VMEM((_CH,), jnp.float32),
            pltpu.VMEM((_CH,), jnp.int32),
            pltpu.VMEM((_CH,), jnp.int32),
            pltpu.VMEM((_CH,), jnp.float32),
            pltpu.VMEM((_CH,), jnp.int32),
            pltpu.VMEM((_CH,), jnp.int32),
            pltpu.VMEM((_CH,), jnp.float32),
            pltpu.VMEM((_CH_T,), jnp.float32),
            pltpu.VMEM((_CH_T,), jnp.int32),
            pltpu.VMEM((_CH_T,), jnp.int32),
            pltpu.VMEM((_CH_T,), jnp.float32),
            pltpu.SemaphoreType.DMA,
            pltpu.SemaphoreType.DMA,
        ],
    )
    return run(qa_pad, dij2d, ii2d, jj2d, zeros, wtab)


_DN_SEG = (((0,), (0,)), ((), ()))


def _moments_body(qa_ref, rx_ref, ry_ref, rz_ref, bs_ref, kcat_ref, mom_ref):
    pid = pl.program_id(0)
    qa = qa_ref[0, 0, :]
    rx = rx_ref[0, 0, :]
    ry = ry_ref[0, 0, :]
    rz = rz_ref[0, 0, :]
    bs = bs_ref[0, 0, :]
    oh = (bs[:, None] == lax.broadcasted_iota(jnp.int32, (_BA, _N_MOL), 1)
          ).astype(jnp.float32)
    hp = jax.lax.Precision.HIGHEST
    kat = jnp.dot(oh, kcat_ref[...], precision=hp)
    kdp = (kat[:, :_NK] * rx[:, None]
           + kat[:, _NK:2 * _NK] * ry[:, None]
           + kat[:, 2 * _NK:] * rz[:, None])
    qc = qa[:, None] * jnp.cos(kdp)
    qs = qa[:, None] * jnp.sin(kdp)
    aidx = pid * _BA + lax.broadcasted_iota(jnp.int32, (_BA,), 0)
    valid = (aidx < _N_ATOMS).astype(jnp.float32)
    mcols = jnp.stack(
        [qa * qa, qa * rx, qa * ry, qa * rz,
         valid, jnp.zeros_like(qa), jnp.zeros_like(qa), jnp.zeros_like(qa)],
        axis=1)
    rcat = jnp.concatenate([qc, qs, mcols], axis=1)
    mom_c = lax.dot_general(oh, rcat, _DN_SEG, precision=hp)

    @pl.when(pid == 0)
    def _():
        mom_ref[...] = jnp.zeros_like(mom_ref)

    mom_ref[...] += mom_c


def _recip_moments_tc(qa2d, rx2d, ry2d, rz2d, bs2d, kcat):
    blk = pl.BlockSpec((1, 1, _BA), lambda i: (i, 0, 0))
    ncols = 2 * _NK + 8
    mom = pl.pallas_call(
        _moments_body,
        grid=(_NBLK,),
        in_specs=[blk, blk, blk, blk, blk,
                  pl.BlockSpec((_N_MOL, 3 * _NK), lambda i: (0, 0))],
        out_specs=pl.BlockSpec((_N_MOL, ncols), lambda i: (0, 0)),
        out_shape=jax.ShapeDtypeStruct((_N_MOL, ncols), jnp.float32),
    )(qa2d, rx2d, ry2d, rz2d, bs2d, kcat)
    return mom[:, :_NK], mom[:, _NK:2 * _NK], mom[:, 2 * _NK:]


def _combine_body(p0_ref, p1_ref, bs_ref, er_ref, out_ref):
    bs = bs_ref[0, 0, :]
    oh = (bs[:, None] == lax.broadcasted_iota(jnp.int32, (_BA, _N_MOL), 1)
          ).astype(jnp.float32)
    er_at = jnp.dot(oh, er_ref[...], precision=jax.lax.Precision.HIGHEST)
    out_ref[0, 0, :] = p0_ref[0, 0, :] + p1_ref[0, 0, :] + er_at[:, 0]


def _combine_tc(p0, p1, bs2d, er_col):
    blk = pl.BlockSpec((1, 1, _BA), lambda i: (i, 0, 0))
    return pl.pallas_call(
        _combine_body,
        grid=(_NBLK,),
        in_specs=[blk, blk, blk, pl.BlockSpec((_N_MOL, 1), lambda i: (0, 0))],
        out_specs=blk,
        out_shape=jax.ShapeDtypeStruct((_NBLK, 1, _BA), jnp.float32),
    )(p0, p1, bs2d, er_col)


def kernel(Z, Dij, Qa, idx_i, idx_j, R, batch_seg, Cell):
    del Z
    f32 = jnp.float32

    qa_pad = jnp.pad(Qa, (0, _NP - _N_ATOMS))

    qa2d = qa_pad.reshape(_NBLK, 1, _BA)
    bs2d = jnp.pad(batch_seg, (0, _NP - _N_ATOMS)).reshape(_NBLK, 1, _BA)
    r_pad = jnp.pad(R, ((0, _NP - _N_ATOMS), (0, 0)))
    rx2d = r_pad[:, 0].reshape(_NBLK, 1, _BA)
    ry2d = r_pad[:, 1].reshape(_NBLK, 1, _BA)
    rz2d = r_pad[:, 2].reshape(_NBLK, 1, _BA)

    c = Cell
    cof00 = c[:, 1, 1] * c[:, 2, 2] - c[:, 1, 2] * c[:, 2, 1]
    cof01 = c[:, 1, 2] * c[:, 2, 0] - c[:, 1, 0] * c[:, 2, 2]
    cof02 = c[:, 1, 0] * c[:, 2, 1] - c[:, 1, 1] * c[:, 2, 0]
    det = (c[:, 0, 0] * cof00 + c[:, 0, 1] * cof01 + c[:, 0, 2] * cof02)
    cof10 = c[:, 0, 2] * c[:, 2, 1] - c[:, 0, 1] * c[:, 2, 2]
    cof11 = c[:, 0, 0] * c[:, 2, 2] - c[:, 0, 2] * c[:, 2, 0]
    cof12 = c[:, 0, 1] * c[:, 2, 0] - c[:, 0, 0] * c[:, 2, 1]
    cof20 = c[:, 0, 1] * c[:, 1, 2] - c[:, 0, 2] * c[:, 1, 1]
    cof21 = c[:, 0, 2] * c[:, 1, 0] - c[:, 0, 0] * c[:, 1, 2]
    cof22 = c[:, 0, 0] * c[:, 1, 1] - c[:, 0, 1] * c[:, 1, 0]
    adj = jnp.stack([
        jnp.stack([cof00, cof01, cof02], axis=-1),
        jnp.stack([cof10, cof11, cof12], axis=-1),
        jnp.stack([cof20, cof21, cof22], axis=-1),
    ], axis=-2)
    recip_box = (2.0 * jnp.pi) * adj / det[:, None, None]
    v_box = jnp.abs(det)
    prefactor = 2.0 * jnp.pi / v_box
    k = jnp.matmul(jnp.asarray(_KVECS), recip_box)
    k2 = jnp.sum(k * k, axis=-1)
    qg = jnp.exp(-0.25 * k2 / _ALPHA2)
    kcat = jnp.concatenate([k[..., 0], k[..., 1], k[..., 2]], axis=1)

    sc_out = _real_space_sc(qa_pad, Dij, idx_i, idx_j)
    q_real, q_imag, misc = _recip_moments_tc(qa2d, rx2d, ry2d, rz2d, bs2d,
                                             kcat.astype(f32))

    qa2_m = misc[:, 0]
    qar = misc[:, 1:4]
    na_m = misc[:, 4]
    q_dens = q_real ** 2 + q_imag ** 2
    e_rec = prefactor * jnp.sum(q_dens * qg / k2)
    e_rec = e_rec - _ALPHA * _ONE_OVER_SQRTPI * qa2_m
    e_rec = e_rec + prefactor / 3.0 * jnp.sum(qar * qar, axis=-1)
    er_col = (e_rec / na_m)[:, None].astype(f32)

    p0 = sc_out[0].reshape(_NBLK, 1, _BA)
    p1 = sc_out[1].reshape(_NBLK, 1, _BA)
    energy2d = _combine_tc(p0, p1, bs2d, er_col)
    energy = energy2d.reshape(_NP)[:_N_ATOMS]
    return (energy, Qa)


# speedup vs baseline: 267.5325x; 1.0003x over previous
import functools
import math

import jax
import jax.numpy as jnp
from jax import lax
from jax.experimental import pallas as pl
from jax.experimental.pallas import tpu as pltpu
from jax.experimental.pallas import tpu_sc as plsc

_CUTOFF = 10.0
_ON_CUT = 0.25 * _CUTOFF
_OFF_CUT = 0.75 * _CUTOFF
_ALPHA = 4.0 / _CUTOFF + 0.001
_ALPHA2 = _ALPHA ** 2
_ONE_OVER_SQRTPI = 1.0 / math.sqrt(math.pi)
_NMAX = (2, 2, 2)

_N_ATOMS = 100000
_N_EDGES = 6400000
_N_MOL = 64

_NC = 2
_NS = 16
_NW = _NC * _NS
_LANES = 16

_NP = 100352
_BA = 2048
_NBLK = _NP // _BA
_CH = 2048
_EPW = _N_EDGES // _NW
_NFULL = _EPW // _CH
_CH_T = _EPW - _NFULL * _CH
_OUT_SLICE = _NP // _NS

_MTAB = 2048
_HTAB = 9.8 / (_MTAB - 1)


def _kvec_mats():
    import itertools
    import numpy as np
    k = []
    for i in range(3):
        kk = [float(v) for v in range(_NMAX[i] + 1)] + [
            float(-v) for v in range(1, _NMAX[i] + 1)]
        k.append(kk)
    lk = list(itertools.product(k[0], k[1], k[2]))[1:]
    kvecs = np.array(lk, dtype=np.float32)
    kmax = max(_NMAX)
    kvecs = kvecs[(kvecs ** 2).sum(-1) <= kmax ** 2]
    return kvecs


_KVECS = _kvec_mats()
_NK = _KVECS.shape[0]


def _radial_weight_table():
    d = jnp.float32(0.2) + _HTAB * jnp.arange(_MTAB, dtype=jnp.float32)
    x = (d - _ON_CUT) / (_OFF_CUT - _ON_CUT)
    ones = jnp.ones_like(x)
    zeros = jnp.zeros_like(x)
    x_p = jnp.where(x <= 0, ones, x)
    fp = jnp.where(x <= 0, zeros, jnp.exp(-ones / x_p))
    x_m = jnp.where(1.0 - x <= 0, ones, 1.0 - x)
    fm = jnp.where(1.0 - x <= 0, zeros, jnp.exp(-ones / x_m))
    f = jnp.where(x <= 0, ones, jnp.where(x >= 1, zeros, fm / (fp + fm)))
    coulomb = 1.0 / d
    damped = 1.0 / jnp.sqrt(d * d + 1.0)
    w = 0.5 * (f * damped + (1.0 - f) * coulomb) * jax.scipy.special.erfc(
        jnp.float32(_ALPHA) * d)
    return w.astype(jnp.float32)


def _compute_chunk(qa_v, wt_v, d_v, ii_v, jj_v, pw_v, nvec):
    inv_h = jnp.float32(1.0 / _HTAB)
    d0 = jnp.float32(0.2)
    imax = jnp.int32(_MTAB - 2)

    @plsc.parallel_loop(0, nvec * _LANES, _LANES, unroll=4)
    def vec_body(co):
        ii = ii_v[pl.ds(co, _LANES)]
        jj = jj_v[pl.ds(co, _LANES)]
        d = d_v[pl.ds(co, _LANES)]
        qi = plsc.load_gather(qa_v, [ii])
        qj = plsc.load_gather(qa_v, [jj])
        x = (d - d0) * inv_h
        i0 = jnp.minimum(x.astype(jnp.int32), imax)
        frac = x - i0.astype(jnp.float32)
        w0 = plsc.load_gather(wt_v, [i0])
        w1 = plsc.load_gather(wt_v, [i0 + 1])
        w = w0 + frac * (w1 - w0)
        pw_v[pl.ds(co, _LANES)] = (qi * qj) * w


def _sc_body(qa_hbm, dij_hbm, ii_hbm, jj_hbm, zeros_hbm, wtab_hbm, out_hbm,
             qa_v, wt_v, acc,
             d0_v, i0_v, j0_v, d1_v, i1_v, j1_v, pw0_v, pw1_v,
             dt_v, it_v, jt_v, pwt_v,
             sem0, sem1, ssem0, ssem1):
    cid = lax.axis_index("c")
    sid = lax.axis_index("s")
    wid = sid * _NC + cid

    pltpu.sync_copy(qa_hbm, qa_v)
    pltpu.sync_copy(wtab_hbm, wt_v)

    @pl.when(sid == 0)
    def _():
        pltpu.sync_copy(zeros_hbm, acc)

    plsc.subcore_barrier()

    e0 = wid * _EPW
    bufs = ((d0_v, i0_v, j0_v, pw0_v, sem0, ssem0),
            (d1_v, i1_v, j1_v, pw1_v, sem1, ssem1))

    def start_loads(ch, buf):
        dv, iv, jv, _, sem, _ = buf
        base = e0 + ch * _CH
        pltpu.async_copy(dij_hbm.at[pl.ds(base, _CH)], dv, sem)
        pltpu.async_copy(ii_hbm.at[pl.ds(base, _CH)], iv, sem)
        pltpu.async_copy(jj_hbm.at[pl.ds(base, _CH)], jv, sem)

    def wait_loads(buf):
        dv, iv, jv, _, sem, _ = buf
        pltpu.make_async_copy(dij_hbm.at[pl.ds(0, _CH)], dv, sem).wait()
        pltpu.make_async_copy(ii_hbm.at[pl.ds(0, _CH)], iv, sem).wait()
        pltpu.make_async_copy(jj_hbm.at[pl.ds(0, _CH)], jv, sem).wait()

    def start_scatter(buf):
        _, iv, _, pwv, _, ssem = buf
        pltpu.async_copy(pwv, acc.at[iv], ssem, add=True)

    def wait_scatter(buf):
        _, iv, _, pwv, _, ssem = buf
        pltpu.make_async_copy(pwv, acc.at[iv], ssem).wait()

    def compute_into(buf):
        dv, iv, jv, pwv, _, _ = buf
        wait_loads(buf)
        _compute_chunk(qa_v, wt_v, dv, iv, jv, pwv, _CH // _LANES)

    start_loads(0, bufs[0])

    def super_body(i, carry):
        ch = i * 2
        @pl.when(i > 0)
        def _():
            wait_scatter(bufs[1])

        start_loads(ch + 1, bufs[1])
        compute_into(bufs[0])
        start_scatter(bufs[0])

        wait_scatter(bufs[0])
        start_loads(ch + 2, bufs[0])
        compute_into(bufs[1])
        start_scatter(bufs[1])
        return carry

    lax.fori_loop(0, (_NFULL - 1) // 2, super_body, 0)
    wait_scatter(bufs[1])
    compute_into(bufs[0])
    pltpu.sync_copy(pw0_v, acc.at[i0_v], add=True)

    tbase = e0 + _NFULL * _CH
    pltpu.sync_copy(dij_hbm.at[pl.ds(tbase, _CH_T)], dt_v)
    pltpu.sync_copy(ii_hbm.at[pl.ds(tbase, _CH_T)], it_v)
    pltpu.sync_copy(jj_hbm.at[pl.ds(tbase, _CH_T)], jt_v)
    _compute_chunk(qa_v, wt_v, dt_v, it_v, jt_v, pwt_v, _CH_T // _LANES)
    pltpu.sync_copy(pwt_v, acc.at[it_v], add=True)

    plsc.subcore_barrier()

    s0 = sid * _OUT_SLICE
    pltpu.sync_copy(acc.at[pl.ds(s0, _OUT_SLICE)],
                    out_hbm.at[cid, pl.ds(s0, _OUT_SLICE)])


def _real_space_sc(qa_pad, dij2d, ii2d, jj2d):
    zeros = jnp.zeros((_NP,), jnp.float32)
    wtab = _radial_weight_table()
    mesh = plsc.VectorSubcoreMesh(core_axis_name="c", subcore_axis_name="s")
    run = pl.kernel(
        _sc_body,
        out_type=jax.ShapeDtypeStruct((_NC, _NP), jnp.float32),
        mesh=mesh,
        compiler_params=pltpu.CompilerParams(
            needs_layout_passes=False, use_tc_tiling_on_sc=False),
        scratch_types=[
            pltpu.VMEM((_NP,), jnp.float32),
            pltpu.VMEM((_MTAB,), jnp.float32),
            pltpu.VMEM_SHARED((_NP,), jnp.float32),
            pltpu.VMEM((_CH,), jnp.float32),
            pltpu.VMEM((_CH,), jnp.int32),
            pltpu.VMEM((_CH,), jnp.int32),
            pltpu.VMEM((_CH,), jnp.float32),
            pltpu.VMEM((_CH,), jnp.int32),
            pltpu.VMEM((_CH,), jnp.int32),
            pltpu.VMEM((_CH,), jnp.float32),
            pltpu.VMEM((_CH,), jnp.float32),
            pltpu.VMEM((_CH_T,), jnp.float32),
            pltpu.VMEM((_CH_T,), jnp.int32),
            pltpu.VMEM((_CH_T,), jnp.int32),
            pltpu.VMEM((_CH_T,), jnp.float32),
            pltpu.SemaphoreType.DMA,
            pltpu.SemaphoreType.DMA,
            pltpu.SemaphoreType.DMA,
            pltpu.SemaphoreType.DMA,
        ],
    )
    return run(qa_pad, dij2d, ii2d, jj2d, zeros, wtab)


_DN_SEG = (((0,), (0,)), ((), ()))


def _moments_body(qa_ref, rx_ref, ry_ref, rz_ref, bs_ref, kcat_ref, mom_ref):
    pid = pl.program_id(0)
    qa = qa_ref[0, 0, :]
    rx = rx_ref[0, 0, :]
    ry = ry_ref[0, 0, :]
    rz = rz_ref[0, 0, :]
    bs = bs_ref[0, 0, :]
    oh = (bs[:, None] == lax.broadcasted_iota(jnp.int32, (_BA, _N_MOL), 1)
          ).astype(jnp.float32)
    hp = jax.lax.Precision.HIGHEST
    kat = jnp.dot(oh, kcat_ref[...], precision=hp)
    kdp = (kat[:, :_NK] * rx[:, None]
           + kat[:, _NK:2 * _NK] * ry[:, None]
           + kat[:, 2 * _NK:] * rz[:, None])
    qc = qa[:, None] * jnp.cos(kdp)
    qs = qa[:, None] * jnp.sin(kdp)
    aidx = pid * _BA + lax.broadcasted_iota(jnp.int32, (_BA,), 0)
    valid = (aidx < _N_ATOMS).astype(jnp.float32)
    mcols = jnp.stack(
        [qa * qa, qa * rx, qa * ry, qa * rz,
         valid, jnp.zeros_like(qa), jnp.zeros_like(qa), jnp.zeros_like(qa)],
        axis=1)
    rcat = jnp.concatenate([qc, qs, mcols], axis=1)
    mom_c = lax.dot_general(oh, rcat, _DN_SEG, precision=hp)

    @pl.when(pid == 0)
    def _():
        mom_ref[...] = jnp.zeros_like(mom_ref)

    mom_ref[...] += mom_c


def _recip_moments_tc(qa2d, rx2d, ry2d, rz2d, bs2d, kcat):
    blk = pl.BlockSpec((1, 1, _BA), lambda i: (i, 0, 0))
    ncols = 2 * _NK + 8
    mom = pl.pallas_call(
        _moments_body,
        grid=(_NBLK,),
        in_specs=[blk, blk, blk, blk, blk,
                  pl.BlockSpec((_N_MOL, 3 * _NK), lambda i: (0, 0))],
        out_specs=pl.BlockSpec((_N_MOL, ncols), lambda i: (0, 0)),
        out_shape=jax.ShapeDtypeStruct((_N_MOL, ncols), jnp.float32),
    )(qa2d, rx2d, ry2d, rz2d, bs2d, kcat)
    return mom[:, :_NK], mom[:, _NK:2 * _NK], mom[:, 2 * _NK:]


def _combine_body(p0_ref, p1_ref, bs_ref, er_ref, out_ref):
    bs = bs_ref[0, 0, :]
    oh = (bs[:, None] == lax.broadcasted_iota(jnp.int32, (_BA, _N_MOL), 1)
          ).astype(jnp.float32)
    er_at = jnp.dot(oh, er_ref[...], precision=jax.lax.Precision.HIGHEST)
    out_ref[0, 0, :] = p0_ref[0, 0, :] + p1_ref[0, 0, :] + er_at[:, 0]


def _combine_tc(p0, p1, bs2d, er_col):
    blk = pl.BlockSpec((1, 1, _BA), lambda i: (i, 0, 0))
    return pl.pallas_call(
        _combine_body,
        grid=(_NBLK,),
        in_specs=[blk, blk, blk, pl.BlockSpec((_N_MOL, 1), lambda i: (0, 0))],
        out_specs=blk,
        out_shape=jax.ShapeDtypeStruct((_NBLK, 1, _BA), jnp.float32),
    )(p0, p1, bs2d, er_col)


def kernel(Z, Dij, Qa, idx_i, idx_j, R, batch_seg, Cell):
    del Z
    f32 = jnp.float32

    qa_pad = jnp.pad(Qa, (0, _NP - _N_ATOMS))

    qa2d = qa_pad.reshape(_NBLK, 1, _BA)
    bs2d = jnp.pad(batch_seg, (0, _NP - _N_ATOMS)).reshape(_NBLK, 1, _BA)
    r_pad = jnp.pad(R, ((0, _NP - _N_ATOMS), (0, 0)))
    rx2d = r_pad[:, 0].reshape(_NBLK, 1, _BA)
    ry2d = r_pad[:, 1].reshape(_NBLK, 1, _BA)
    rz2d = r_pad[:, 2].reshape(_NBLK, 1, _BA)

    c = Cell
    cof00 = c[:, 1, 1] * c[:, 2, 2] - c[:, 1, 2] * c[:, 2, 1]
    cof01 = c[:, 1, 2] * c[:, 2, 0] - c[:, 1, 0] * c[:, 2, 2]
    cof02 = c[:, 1, 0] * c[:, 2, 1] - c[:, 1, 1] * c[:, 2, 0]
    det = (c[:, 0, 0] * cof00 + c[:, 0, 1] * cof01 + c[:, 0, 2] * cof02)
    cof10 = c[:, 0, 2] * c[:, 2, 1] - c[:, 0, 1] * c[:, 2, 2]
    cof11 = c[:, 0, 0] * c[:, 2, 2] - c[:, 0, 2] * c[:, 2, 0]
    cof12 = c[:, 0, 1] * c[:, 2, 0] - c[:, 0, 0] * c[:, 2, 1]
    cof20 = c[:, 0, 1] * c[:, 1, 2] - c[:, 0, 2] * c[:, 1, 1]
    cof21 = c[:, 0, 2] * c[:, 1, 0] - c[:, 0, 0] * c[:, 1, 2]
    cof22 = c[:, 0, 0] * c[:, 1, 1] - c[:, 0, 1] * c[:, 1, 0]
    adj = jnp.stack([
        jnp.stack([cof00, cof01, cof02], axis=-1),
        jnp.stack([cof10, cof11, cof12], axis=-1),
        jnp.stack([cof20, cof21, cof22], axis=-1),
    ], axis=-2)
    recip_box = (2.0 * jnp.pi) * adj / det[:, None, None]
    v_box = jnp.abs(det)
    prefactor = 2.0 * jnp.pi / v_box
    k = jnp.matmul(jnp.asarray(_KVECS), recip_box)
    k2 = jnp.sum(k * k, axis=-1)
    qg = jnp.exp(-0.25 * k2 / _ALPHA2)
    kcat = jnp.concatenate([k[..., 0], k[..., 1], k[..., 2]], axis=1)

    sc_out = _real_space_sc(qa_pad, Dij, idx_i, idx_j)
    q_real, q_imag, misc = _recip_moments_tc(qa2d, rx2d, ry2d, rz2d, bs2d,
                                             kcat.astype(f32))

    qa2_m = misc[:, 0]
    qar = misc[:, 1:4]
    na_m = misc[:, 4]
    q_dens = q_real ** 2 + q_imag ** 2
    e_rec = prefactor * jnp.sum(q_dens * qg / k2)
    e_rec = e_rec - _ALPHA * _ONE_OVER_SQRTPI * qa2_m
    e_rec = e_rec + prefactor / 3.0 * jnp.sum(qar * qar, axis=-1)
    er_col = (e_rec / na_m)[:, None].astype(f32)

    p0 = sc_out[0].reshape(_NBLK, 1, _BA)
    p1 = sc_out[1].reshape(_NBLK, 1, _BA)
    energy2d = _combine_tc(p0, p1, bs2d, er_col)
    energy = energy2d.reshape(_NP)[:_N_ATOMS]
    return (energy, Qa)
